# baseline (speedup 1.0000x reference)
import functools

import jax
import jax.numpy as jnp
from jax import lax
from jax.experimental import pallas as pl
from jax.experimental.pallas import tpu as pltpu
from jax.experimental.pallas import tpu_sc as plsc

_LANES = 16
_RPB = 8


def _sc_gather(table, idxs, V, B, NJ):
    nblk = -(-NJ // _RPB)
    nwork = 3 * nblk
    tail = NJ - (nblk - 1) * _RPB
    NQ = 4
    Q = B // NQ

    mesh = plsc.VectorSubcoreMesh(core_axis_name="c", subcore_axis_name="s",
                                  num_cores=1)

    @functools.partial(
        pl.kernel,
        mesh=mesh,
        compiler_params=pltpu.CompilerParams(use_tc_tiling_on_sc=True),
        out_type=jax.ShapeDtypeStruct((3, NJ, B), jnp.float32),
        scratch_types=[
            pltpu.VMEM((24,), jnp.int32),
            pltpu.VMEM((_RPB, B), jnp.float32),
        ] + [pltpu.SemaphoreType.DMA] * (NQ + 1),
    )
    def k(table_hbm, idx_hbm, out_hbm, jnt_v, rows_v, *sems):
        sem_o = sems[NQ]
        wid = lax.axis_index("s") + lax.axis_index("c")

        @pl.when(wid < nwork)
        def _():
            cidx = lax.shift_right_logical(wid * 21846, 16)
            blk = wid - cidx * 3
            jnt_v[pl.ds(_RPB, _LANES)] = jnp.zeros((_LANES,), jnp.int32)
            pltpu.sync_copy(idx_hbm, jnt_v.at[pl.ds(0, NJ)])
            win = jnt_v.at[pl.ds(blk * _RPB, _RPB)]
            src = table_hbm.at[cidx]
            dst = out_hbm.at[cidx]

            def _move(nrows, row0):
                gathers = [
                    pltpu.async_copy(
                        src.at[win, pl.ds(h * Q, Q)],
                        rows_v.at[pl.ds(0, _RPB), pl.ds(h * Q, Q)], sems[h])
                    for h in range(NQ)
                ]
                outs = []
                for h in range(NQ):
                    gathers[h].wait()
                    outs.append(pltpu.async_copy(
                        rows_v.at[pl.ds(0, nrows), pl.ds(h * Q, Q)],
                        dst.at[pl.ds(row0, nrows), pl.ds(h * Q, Q)], sem_o))
                for o in outs:
                    o.wait()

            @pl.when(blk < nblk - 1)
            def _():
                _move(_RPB, blk * _RPB)

            @pl.when(blk == nblk - 1)
            def _():
                _move(tail, (nblk - 1) * _RPB)

    return k(table, idxs)


def kernel(vertices, extra_joints_idxs):
    B, V, C = vertices.shape
    NJ = extra_joints_idxs.shape[0]
    assert C == 3 and NJ == 21 and B % 256 == 0
    table = jnp.transpose(vertices, (2, 1, 0))
    out_t = _sc_gather(table, extra_joints_idxs.astype(jnp.int32), V, B, NJ)
    return jnp.transpose(out_t, (2, 1, 0))


# speedup vs baseline: 1.0107x; 1.0107x over previous
import functools

import jax
import jax.numpy as jnp
from jax import lax
from jax.experimental import pallas as pl
from jax.experimental.pallas import tpu as pltpu
from jax.experimental.pallas import tpu_sc as plsc

_LANES = 16
_RPB = 8


def _sc_gather(table, idxs, V, B, NJ):
    nblk = -(-NJ // _RPB)
    nwork = 3 * nblk
    tail = NJ - (nblk - 1) * _RPB
    NQ = 2
    Q = B // NQ

    mesh = plsc.VectorSubcoreMesh(core_axis_name="c", subcore_axis_name="s",
                                  num_cores=1)

    @functools.partial(
        pl.kernel,
        mesh=mesh,
        compiler_params=pltpu.CompilerParams(use_tc_tiling_on_sc=True),
        out_type=jax.ShapeDtypeStruct((3, NJ, B), jnp.float32),
        scratch_types=[
            pltpu.VMEM((24,), jnp.int32),
            pltpu.VMEM((_RPB, B), jnp.float32),
        ] + [pltpu.SemaphoreType.DMA] * (NQ + 1),
    )
    def k(table_hbm, idx_hbm, out_hbm, jnt_v, rows_v, *sems):
        sem_o = sems[NQ]
        wid = lax.axis_index("s") + lax.axis_index("c")

        @pl.when(wid < nwork)
        def _():
            cidx = lax.shift_right_logical(wid * 21846, 16)
            blk = wid - cidx * 3
            jnt_v[pl.ds(_RPB, _LANES)] = jnp.zeros((_LANES,), jnp.int32)
            pltpu.sync_copy(idx_hbm, jnt_v.at[pl.ds(0, NJ)])
            win = jnt_v.at[pl.ds(blk * _RPB, _RPB)]
            src = table_hbm.at[cidx]
            dst = out_hbm.at[cidx]

            def _move(nrows, row0):
                gathers = [
                    pltpu.async_copy(
                        src.at[win, pl.ds(h * Q, Q)],
                        rows_v.at[pl.ds(0, _RPB), pl.ds(h * Q, Q)], sems[h])
                    for h in range(NQ)
                ]
                outs = []
                for h in range(NQ):
                    gathers[h].wait()
                    outs.append(pltpu.async_copy(
                        rows_v.at[pl.ds(0, nrows), pl.ds(h * Q, Q)],
                        dst.at[pl.ds(row0, nrows), pl.ds(h * Q, Q)], sem_o))
                for o in outs:
                    o.wait()

            @pl.when(blk < nblk - 1)
            def _():
                _move(_RPB, blk * _RPB)

            @pl.when(blk == nblk - 1)
            def _():
                _move(tail, (nblk - 1) * _RPB)

    return k(table, idxs)


def kernel(vertices, extra_joints_idxs):
    B, V, C = vertices.shape
    NJ = extra_joints_idxs.shape[0]
    assert C == 3 and NJ == 21 and B % 256 == 0
    table = jnp.transpose(vertices, (2, 1, 0))
    out_t = _sc_gather(table, extra_joints_idxs.astype(jnp.int32), V, B, NJ)
    return jnp.transpose(out_t, (2, 1, 0))
